# baseline (speedup 1.0000x reference)
import functools

import jax
import jax.numpy as jnp
from jax.experimental import pallas as pl
from jax.experimental.pallas import tpu as pltpu

INPUT_DIM = 2048
WIDTH = 16384
K = 64


def _enc_body(x_ref, aet_ref, be_ref, h_ref):
    h_ref[...] = (
        jnp.dot(x_ref[...], aet_ref[...], preferred_element_type=jnp.float32)
        + be_ref[...]
    )


def _dec_body(c_ref, adt_ref, bd_ref, o_ref):
    k = pl.program_id(2)

    @pl.when(k == 0)
    def _():
        o_ref[...] = jnp.broadcast_to(bd_ref[...], o_ref.shape)

    o_ref[...] += jnp.dot(
        c_ref[...], adt_ref[...], preferred_element_type=jnp.float32
    )


@functools.partial(jax.jit, static_argnames=())
def kernel(x, Ae, be, bd, Ad):
    n = x.shape[0]
    xc = x - bd
    AeT = Ae.T

    bm, bn = 256, 1024
    h = pl.pallas_call(
        _enc_body,
        grid=(n // bm, WIDTH // bn),
        in_specs=[
            pl.BlockSpec((bm, INPUT_DIM), lambda i, j: (i, 0)),
            pl.BlockSpec((INPUT_DIM, bn), lambda i, j: (0, j)),
            pl.BlockSpec((1, bn), lambda i, j: (0, j)),
        ],
        out_specs=pl.BlockSpec((bm, bn), lambda i, j: (i, j)),
        out_shape=jax.ShapeDtypeStruct((n, WIDTH), jnp.float32),
    )(xc, AeT, be)

    topk_values, topk_indices = jax.lax.top_k(h, K)
    row_idx = jnp.arange(n)[:, None]
    codes = jnp.zeros_like(h).at[row_idx, topk_indices].set(
        jax.nn.relu(topk_values)
    )

    AdT = Ad.T
    bm2, bn2, bk2 = 512, 1024, 2048
    out = pl.pallas_call(
        _dec_body,
        grid=(n // bm2, INPUT_DIM // bn2, WIDTH // bk2),
        in_specs=[
            pl.BlockSpec((bm2, bk2), lambda i, j, k: (i, k)),
            pl.BlockSpec((bk2, bn2), lambda i, j, k: (k, j)),
            pl.BlockSpec((1, bn2), lambda i, j, k: (0, j)),
        ],
        out_specs=pl.BlockSpec((bm2, bn2), lambda i, j, k: (i, j)),
        out_shape=jax.ShapeDtypeStruct((n, INPUT_DIM), jnp.float32),
        compiler_params=pltpu.CompilerParams(
            dimension_semantics=("parallel", "parallel", "arbitrary"),
        ),
    )(codes, AdT, bd)
    return out


# speedup vs baseline: 3.6134x; 3.6134x over previous
import functools

import jax
import jax.numpy as jnp
from jax import lax
from jax.experimental import pallas as pl
from jax.experimental.pallas import tpu as pltpu
from jax.experimental.pallas import tpu_sc as plsc

INPUT_DIM = 2048
WIDTH = 16384
NTOK = 2048
K = 64
L = 16
NW = 32
ROWS_PER_W = NTOK // NW
NVREG = WIDTH // L
UN = 8


def _enc_body(x_ref, aet_ref, be_ref, h_ref):
    h_ref[...] = (
        jnp.dot(x_ref[...], aet_ref[...], preferred_element_type=jnp.float32)
        + be_ref[...]
    )


def _dec_body(c_ref, adt_ref, bd_ref, o_ref):
    k = pl.program_id(2)

    @pl.when(k == 0)
    def _():
        o_ref[...] = jnp.broadcast_to(bd_ref[...], o_ref.shape)

    o_ref[...] += jnp.dot(
        c_ref[...], adt_ref[...], preferred_element_type=jnp.float32
    )


def _sortable(f):
    b = lax.bitcast_convert_type(f, jnp.int32)
    return b ^ (lax.shift_right_arithmetic(b, 31) & jnp.int32(0x7FFFFFFF))


def _sc_body(h_hbm, codes_hbm, hrow, cand_s, cand_i, hist, tot,
             sel_a, sel_b, crow_a, crow_b, sem_a, sem_b):
    lanes = lax.iota(jnp.int32, L)
    lanebase = lanes * 256
    ones = jnp.ones((L,), jnp.int32)
    zf = jnp.zeros((L,), jnp.float32)

    wid = lax.axis_index("s") * 2 + lax.axis_index("c")
    base = wid * ROWS_PER_W

    def zero_hist():
        def zh(v, _):
            for u in range(UN):
                hist[pl.ds((v * UN + u) * L, L)] = jnp.zeros((L,), jnp.int32)
            return 0
        lax.fori_loop(0, 4096 // (L * UN), zh, 0)

    def reduce_hist():
        def rc(c, _):
            acc = jnp.zeros((L,), jnp.int32)
            for l in range(L):
                acc = acc + hist[pl.ds(l * 256 + c * L, L)]
            tot[pl.ds(c * L, L)] = acc
            return 0
        lax.fori_loop(0, 16, rc, 0)

    def boundary_find(need):
        def bc(c, csum):
            t = tot[pl.ds(c * L, L)]
            return jnp.where(lanes == c, jnp.sum(t), csum)
        csum = lax.fori_loop(0, 16, bc, jnp.zeros((L,), jnp.int32))
        sfx_c = lax.rev(jnp.cumsum(lax.rev(csum, (0,))), (0,))
        cstar = jnp.sum((sfx_c >= need).astype(jnp.int32)) - 1
        prefix_above = jnp.sum(jnp.where(lanes == cstar, sfx_c - csum, 0))
        t_chunk = tot[pl.ds(cstar * L, L)]
        sfx2 = lax.rev(jnp.cumsum(lax.rev(t_chunk, (0,))), (0,)) + prefix_above
        l1 = jnp.sum((sfx2 >= need).astype(jnp.int32)) - 1
        b = cstar * L + l1
        g = jnp.sum(jnp.where(lanes == l1, sfx2 - t_chunk, 0))
        return b, need - g

    def row_body(r, _):
        row = base + r
        pltpu.sync_copy(h_hbm.at[row], hrow)

        zero_hist()

        def h1(v, _):
            for u in range(UN):
                i = v * UN + u
                s = _sortable(hrow[pl.ds(i * L, L)])
                bkt = lax.shift_right_arithmetic(s, 24) + 128
                plsc.addupdate_scatter(hist, [lanebase + bkt], ones)
            return 0
        lax.fori_loop(0, NVREG // UN, h1, 0)
        reduce_hist()
        b1, need = boundary_find(K)

        zero_hist()

        def h2(v, pos):
            for u in range(UN):
                i = v * UN + u
                s = _sortable(hrow[pl.ds(i * L, L)])
                bkt = lax.shift_right_arithmetic(s, 24) + 128
                m = bkt >= b1
                plsc.store_compressed(cand_s.at[pl.ds(pos, L)], s, mask=m)
                plsc.store_compressed(
                    cand_i.at[pl.ds(pos, L)], i * L + lanes, mask=m)
                meq = bkt == b1
                b2v = lax.shift_right_arithmetic(s, 16) & 0xFF
                plsc.addupdate_scatter(
                    hist, [lanebase + b2v], ones, mask=meq)
                pos = pos + jnp.sum(m.astype(jnp.int32))
            return pos
        cnt = lax.fori_loop(0, NVREG // UN, h2, jnp.int32(0))
        reduce_hist()
        b2, need = boundary_find(need)
        p16 = (b1 - 128) * 256 + b2

        nv = lax.div(cnt + (L - 1), L)

        zero_hist()

        def h3(v, _):
            sv = cand_s[pl.ds(v * L, L)]
            valid = (v * L + lanes) < cnt
            m = valid & (lax.shift_right_arithmetic(sv, 16) == p16)
            b3v = lax.shift_right_arithmetic(sv, 8) & 0xFF
            plsc.addupdate_scatter(hist, [lanebase + b3v], ones, mask=m)
            return 0
        lax.fori_loop(0, nv, h3, 0)
        reduce_hist()
        b3, need = boundary_find(need)
        p24 = p16 * 256 + b3

        zero_hist()

        def h4(v, _):
            sv = cand_s[pl.ds(v * L, L)]
            valid = (v * L + lanes) < cnt
            m = valid & (lax.shift_right_arithmetic(sv, 8) == p24)
            b4v = sv & 0xFF
            plsc.addupdate_scatter(hist, [lanebase + b4v], ones, mask=m)
            return 0
        lax.fori_loop(0, nv, h4, 0)
        reduce_hist()
        b4, need_final = boundary_find(need)
        t_s = p24 * 256 + b4

        def emit(crow, selref, sem):
            @pl.when(r >= 2)
            def _():
                pltpu.make_async_copy(crow, codes_hbm.at[row - 2], sem).wait()
                for j in range(K // L):
                    zi = selref[pl.ds(j * L, L)]
                    plsc.store_scatter(crow, [zi], zf)

            def ap(v, carry):
                pos2, eqc = carry
                sv = cand_s[pl.ds(v * L, L)]
                iv = cand_i[pl.ds(v * L, L)]
                valid = (v * L + lanes) < cnt
                gt = valid & (sv > t_s)
                eq = valid & (sv == t_s)
                eqi = eq.astype(jnp.int32)
                rank = eqc + jnp.cumsum(eqi) - eqi
                take = gt | (eq & (rank < need_final))
                fv = lax.bitcast_convert_type(
                    sv ^ (lax.shift_right_arithmetic(sv, 31)
                          & jnp.int32(0x7FFFFFFF)),
                    jnp.float32)
                val = jnp.maximum(fv, 0.0)
                plsc.store_scatter(crow, [iv], val, mask=take)
                plsc.store_compressed(
                    selref.at[pl.ds(pos2, L)], iv, mask=take)
                return (pos2 + jnp.sum(take.astype(jnp.int32)),
                        eqc + jnp.sum(eqi))
            lax.fori_loop(0, nv, ap, (jnp.int32(0), jnp.int32(0)))
            pltpu.async_copy(crow, codes_hbm.at[row], sem)

        @pl.when(r % 2 == 0)
        def _():
            emit(crow_a, sel_a, sem_a)

        @pl.when(r % 2 == 1)
        def _():
            emit(crow_b, sel_b, sem_b)
        return 0

    def zb(i, _):
        crow_a[pl.ds(i * L, L)] = zf
        crow_b[pl.ds(i * L, L)] = zf
        return 0
    lax.fori_loop(0, NVREG, zb, 0)

    lax.fori_loop(0, ROWS_PER_W, row_body, 0)

    pltpu.make_async_copy(
        crow_a, codes_hbm.at[base + ROWS_PER_W - 2], sem_a).wait()
    pltpu.make_async_copy(
        crow_b, codes_hbm.at[base + ROWS_PER_W - 1], sem_b).wait()


def _sc_topk_codes(h):
    mesh = plsc.VectorSubcoreMesh(
        core_axis_name="c", subcore_axis_name="s", num_cores=2)
    f = pl.kernel(
        _sc_body,
        out_type=jax.ShapeDtypeStruct((NTOK, WIDTH), jnp.float32),
        mesh=mesh,
        compiler_params=pltpu.CompilerParams(needs_layout_passes=False),
        scratch_types=[
            pltpu.VMEM((WIDTH,), jnp.float32),
            pltpu.VMEM((WIDTH + L,), jnp.int32),
            pltpu.VMEM((WIDTH + L,), jnp.int32),
            pltpu.VMEM((256 * L,), jnp.int32),
            pltpu.VMEM((256,), jnp.int32),
            pltpu.VMEM((K + L,), jnp.int32),
            pltpu.VMEM((K + L,), jnp.int32),
            pltpu.VMEM((WIDTH,), jnp.float32),
            pltpu.VMEM((WIDTH,), jnp.float32),
            pltpu.SemaphoreType.DMA,
            pltpu.SemaphoreType.DMA,
        ],
    )
    return f(h)


@jax.jit
def kernel(x, Ae, be, bd, Ad):
    n = x.shape[0]
    xc = x - bd
    AeT = Ae.T

    bm, bn = 256, 1024
    h = pl.pallas_call(
        _enc_body,
        grid=(n // bm, WIDTH // bn),
        in_specs=[
            pl.BlockSpec((bm, INPUT_DIM), lambda i, j: (i, 0)),
            pl.BlockSpec((INPUT_DIM, bn), lambda i, j: (0, j)),
            pl.BlockSpec((1, bn), lambda i, j: (0, j)),
        ],
        out_specs=pl.BlockSpec((bm, bn), lambda i, j: (i, j)),
        out_shape=jax.ShapeDtypeStruct((n, WIDTH), jnp.float32),
    )(xc, AeT, be)

    codes = _sc_topk_codes(h)

    AdT = Ad.T
    bm2, bn2, bk2 = 512, 1024, 2048
    out = pl.pallas_call(
        _dec_body,
        grid=(n // bm2, INPUT_DIM // bn2, WIDTH // bk2),
        in_specs=[
            pl.BlockSpec((bm2, bk2), lambda i, j, k: (i, k)),
            pl.BlockSpec((bk2, bn2), lambda i, j, k: (k, j)),
            pl.BlockSpec((1, bn2), lambda i, j, k: (0, j)),
        ],
        out_specs=pl.BlockSpec((bm2, bn2), lambda i, j, k: (i, j)),
        out_shape=jax.ShapeDtypeStruct((n, INPUT_DIM), jnp.float32),
        compiler_params=pltpu.CompilerParams(
            dimension_semantics=("parallel", "parallel", "arbitrary"),
        ),
    )(codes, AdT, bd)
    return out


# speedup vs baseline: 4.0277x; 1.1147x over previous
import functools

import jax
import jax.numpy as jnp
from jax import lax
from jax.experimental import pallas as pl
from jax.experimental.pallas import tpu as pltpu
from jax.experimental.pallas import tpu_sc as plsc

INPUT_DIM = 2048
WIDTH = 16384
NTOK = 2048
K = 64
L = 16
NW = 32
ROWS_PER_W = NTOK // NW
NVREG = WIDTH // L
UN = 8

_CONTRACT_LAST = (((1,), (1,)), ((), ()))


def _enc_body(x_ref, ae_ref, be_ref, h_ref):
    h_ref[...] = (
        lax.dot_general(x_ref[...], ae_ref[...], _CONTRACT_LAST,
                        preferred_element_type=jnp.float32)
        + be_ref[...]
    )


def _dec_body(c_ref, ad_ref, bd_ref, o_ref):
    k = pl.program_id(2)

    @pl.when(k == 0)
    def _():
        o_ref[...] = jnp.broadcast_to(bd_ref[...], o_ref.shape)

    o_ref[...] += lax.dot_general(c_ref[...], ad_ref[...], _CONTRACT_LAST,
                                  preferred_element_type=jnp.float32)


def _sortable(f):
    b = lax.bitcast_convert_type(f, jnp.int32)
    return b ^ (lax.shift_right_arithmetic(b, 31) & jnp.int32(0x7FFFFFFF))


def _sc_body(h_hbm, codes_hbm, hrow_a, hrow_b, cand_s, cand_i, hist, tot,
             sel_a, sel_b, crow_a, crow_b, sem_a, sem_b, sem_ha, sem_hb):
    lanes = lax.iota(jnp.int32, L)
    lanebase = lanes * 256
    ones = jnp.ones((L,), jnp.int32)
    zi16 = jnp.zeros((L,), jnp.int32)
    zf = jnp.zeros((L,), jnp.float32)

    wid = lax.axis_index("s") * 2 + lax.axis_index("c")
    base = wid * ROWS_PER_W

    def reduce_and_clear_hist():
        def rc(c, csum):
            vals = [hist[pl.ds(l * 256 + c * L, L)] for l in range(L)]
            for l in range(L):
                hist[pl.ds(l * 256 + c * L, L)] = zi16
            while len(vals) > 1:
                vals = [vals[2 * i] + vals[2 * i + 1]
                        for i in range(len(vals) // 2)]
            acc = vals[0]
            tot[pl.ds(c * L, L)] = acc
            return jnp.where(lanes == c, jnp.sum(acc), csum)
        return lax.fori_loop(0, 16, rc, zi16)

    def boundary_find(csum, need):
        sfx_c = lax.rev(jnp.cumsum(lax.rev(csum, (0,))), (0,))
        cstar = jnp.sum((sfx_c >= need).astype(jnp.int32)) - 1
        prefix_above = jnp.sum(jnp.where(lanes == cstar, sfx_c - csum, 0))
        t_chunk = tot[pl.ds(cstar * L, L)]
        sfx2 = lax.rev(jnp.cumsum(lax.rev(t_chunk, (0,))), (0,)) + prefix_above
        l1 = jnp.sum((sfx2 >= need).astype(jnp.int32)) - 1
        b = cstar * L + l1
        g = jnp.sum(jnp.where(lanes == l1, sfx2 - t_chunk, 0))
        return b, need - g

    def work(row, hrow, crow, selref, sem, not_first):
        def h1(v, _):
            for u in range(UN):
                i = v * UN + u
                s = _sortable(hrow[pl.ds(i * L, L)])
                bkt = lax.shift_right_arithmetic(s, 24) + 128
                plsc.addupdate_scatter(hist, [lanebase + bkt], ones)
            return 0
        lax.fori_loop(0, NVREG // UN, h1, 0)
        b1, need = boundary_find(reduce_and_clear_hist(), K)

        def h2(v, pos):
            for u in range(UN):
                i = v * UN + u
                s = _sortable(hrow[pl.ds(i * L, L)])
                bkt = lax.shift_right_arithmetic(s, 24) + 128
                m = bkt >= b1
                plsc.store_compressed(cand_s.at[pl.ds(pos, L)], s, mask=m)
                plsc.store_compressed(
                    cand_i.at[pl.ds(pos, L)], i * L + lanes, mask=m)
                meq = bkt == b1
                b2v = lax.shift_right_arithmetic(s, 16) & 0xFF
                plsc.addupdate_scatter(
                    hist, [lanebase + b2v], ones, mask=meq)
                pos = pos + jnp.sum(m.astype(jnp.int32))
            return pos
        cnt = lax.fori_loop(0, NVREG // UN, h2, jnp.int32(0))
        b2, need = boundary_find(reduce_and_clear_hist(), need)
        p16 = (b1 - 128) * 256 + b2

        nv = lax.div(cnt + (L - 1), L)

        def h3(v, _):
            sv = cand_s[pl.ds(v * L, L)]
            valid = (v * L + lanes) < cnt
            m = valid & (lax.shift_right_arithmetic(sv, 16) == p16)
            b3v = lax.shift_right_arithmetic(sv, 8) & 0xFF
            plsc.addupdate_scatter(hist, [lanebase + b3v], ones, mask=m)
            return 0
        lax.fori_loop(0, nv, h3, 0)
        b3, need = boundary_find(reduce_and_clear_hist(), need)
        p24 = p16 * 256 + b3

        def h4(v, _):
            sv = cand_s[pl.ds(v * L, L)]
            valid = (v * L + lanes) < cnt
            m = valid & (lax.shift_right_arithmetic(sv, 8) == p24)
            plsc.addupdate_scatter(hist, [lanebase + (sv & 0xFF)], ones,
                                   mask=m)
            return 0
        lax.fori_loop(0, nv, h4, 0)
        b4, need_final = boundary_find(reduce_and_clear_hist(), need)
        t_s = p24 * 256 + b4

        @pl.when(not_first)
        def _():
            pltpu.make_async_copy(crow, codes_hbm.at[row - 2], sem).wait()
            for j in range(K // L):
                zi = selref[pl.ds(j * L, L)]
                plsc.store_scatter(crow, [zi], zf)

        def ap(v, carry):
            pos2, eqc = carry
            sv = cand_s[pl.ds(v * L, L)]
            iv = cand_i[pl.ds(v * L, L)]
            valid = (v * L + lanes) < cnt
            gt = valid & (sv > t_s)
            eq = valid & (sv == t_s)
            eqi = eq.astype(jnp.int32)
            rank = eqc + jnp.cumsum(eqi) - eqi
            take = gt | (eq & (rank < need_final))
            fv = lax.bitcast_convert_type(
                sv ^ (lax.shift_right_arithmetic(sv, 31)
                      & jnp.int32(0x7FFFFFFF)),
                jnp.float32)
            val = jnp.maximum(fv, 0.0)
            plsc.store_scatter(crow, [iv], val, mask=take)
            plsc.store_compressed(selref.at[pl.ds(pos2, L)], iv, mask=take)
            return (pos2 + jnp.sum(take.astype(jnp.int32)),
                    eqc + jnp.sum(eqi))
        lax.fori_loop(0, nv, ap, (jnp.int32(0), jnp.int32(0)))
        pltpu.async_copy(crow, codes_hbm.at[row], sem)

    def zb(i, _):
        crow_a[pl.ds(i * L, L)] = zf
        crow_b[pl.ds(i * L, L)] = zf
        return 0
    lax.fori_loop(0, NVREG, zb, 0)

    def zh(i, _):
        hist[pl.ds(i * L, L)] = zi16
        return 0
    lax.fori_loop(0, 256, zh, 0)

    pltpu.async_copy(h_hbm.at[base], hrow_a, sem_ha)

    def pair_body(i, _):
        row0 = base + 2 * i
        pltpu.async_copy(h_hbm.at[row0 + 1], hrow_b, sem_hb)
        pltpu.make_async_copy(h_hbm.at[row0], hrow_a, sem_ha).wait()
        work(row0, hrow_a, crow_a, sel_a, sem_a, i >= 1)

        @pl.when(i < ROWS_PER_W // 2 - 1)
        def _():
            pltpu.async_copy(h_hbm.at[row0 + 2], hrow_a, sem_ha)

        pltpu.make_async_copy(h_hbm.at[row0 + 1], hrow_b, sem_hb).wait()
        work(row0 + 1, hrow_b, crow_b, sel_b, sem_b, i >= 1)
        return 0
    lax.fori_loop(0, ROWS_PER_W // 2, pair_body, 0)

    pltpu.make_async_copy(
        crow_a, codes_hbm.at[base + ROWS_PER_W - 2], sem_a).wait()
    pltpu.make_async_copy(
        crow_b, codes_hbm.at[base + ROWS_PER_W - 1], sem_b).wait()


def _sc_topk_codes(h):
    mesh = plsc.VectorSubcoreMesh(
        core_axis_name="c", subcore_axis_name="s", num_cores=2)
    f = pl.kernel(
        _sc_body,
        out_type=jax.ShapeDtypeStruct((NTOK, WIDTH), jnp.float32),
        mesh=mesh,
        compiler_params=pltpu.CompilerParams(needs_layout_passes=False),
        scratch_types=[
            pltpu.VMEM((WIDTH,), jnp.float32),
            pltpu.VMEM((WIDTH,), jnp.float32),
            pltpu.VMEM((WIDTH + L,), jnp.int32),
            pltpu.VMEM((WIDTH + L,), jnp.int32),
            pltpu.VMEM((256 * L,), jnp.int32),
            pltpu.VMEM((256,), jnp.int32),
            pltpu.VMEM((K + L,), jnp.int32),
            pltpu.VMEM((K + L,), jnp.int32),
            pltpu.VMEM((WIDTH,), jnp.float32),
            pltpu.VMEM((WIDTH,), jnp.float32),
            pltpu.SemaphoreType.DMA,
            pltpu.SemaphoreType.DMA,
            pltpu.SemaphoreType.DMA,
            pltpu.SemaphoreType.DMA,
        ],
    )
    return f(h)


@jax.jit
def kernel(x, Ae, be, bd, Ad):
    n = x.shape[0]
    xc = x - bd

    bm, bn = 256, 1024
    h = pl.pallas_call(
        _enc_body,
        grid=(n // bm, WIDTH // bn),
        in_specs=[
            pl.BlockSpec((bm, INPUT_DIM), lambda i, j: (i, 0)),
            pl.BlockSpec((bn, INPUT_DIM), lambda i, j: (j, 0)),
            pl.BlockSpec((1, bn), lambda i, j: (0, j)),
        ],
        out_specs=pl.BlockSpec((bm, bn), lambda i, j: (i, j)),
        out_shape=jax.ShapeDtypeStruct((n, WIDTH), jnp.float32),
    )(xc, Ae, be)

    codes = _sc_topk_codes(h)

    bm2, bn2, bk2 = 512, 1024, 2048
    out = pl.pallas_call(
        _dec_body,
        grid=(n // bm2, INPUT_DIM // bn2, WIDTH // bk2),
        in_specs=[
            pl.BlockSpec((bm2, bk2), lambda i, j, k: (i, k)),
            pl.BlockSpec((bn2, bk2), lambda i, j, k: (j, k)),
            pl.BlockSpec((1, bn2), lambda i, j, k: (0, j)),
        ],
        out_specs=pl.BlockSpec((bm2, bn2), lambda i, j, k: (i, j)),
        out_shape=jax.ShapeDtypeStruct((n, INPUT_DIM), jnp.float32),
        compiler_params=pltpu.CompilerParams(
            dimension_semantics=("parallel", "parallel", "arbitrary"),
        ),
    )(codes, Ad, bd)
    return out


# speedup vs baseline: 4.2505x; 1.0553x over previous
import functools

import jax
import jax.numpy as jnp
from jax import lax
from jax.experimental import pallas as pl
from jax.experimental.pallas import tpu as pltpu
from jax.experimental.pallas import tpu_sc as plsc

INPUT_DIM = 2048
WIDTH = 16384
NTOK = 2048
K = 64
L = 16
NW = 32
ROWS_PER_W = NTOK // NW
NVREG = WIDTH // L
UN = 8

_CONTRACT_LAST = (((1,), (1,)), ((), ()))


def _enc_body(x_ref, ae_ref, be_ref, h_ref):
    h_ref[...] = (
        lax.dot_general(x_ref[...], ae_ref[...], _CONTRACT_LAST,
                        preferred_element_type=jnp.float32)
        + be_ref[...]
    )


def _dec_body(c_ref, ad_ref, bd_ref, o_ref):
    k = pl.program_id(2)

    @pl.when(k == 0)
    def _():
        o_ref[...] = jnp.broadcast_to(bd_ref[...], o_ref.shape)

    o_ref[...] += lax.dot_general(c_ref[...], ad_ref[...], _CONTRACT_LAST,
                                  preferred_element_type=jnp.float32)


def _sortable(f):
    b = lax.bitcast_convert_type(f, jnp.int32)
    return b ^ (lax.shift_right_arithmetic(b, 31) & jnp.int32(0x7FFFFFFF))


def _sc_body(h_hbm, codes_hbm, hrow_a, hrow_b, cand_s, cand_i, hist, tot,
             sel_a, sel_b, crow_a, crow_b, sem_a, sem_b, sem_ha, sem_hb):
    lanes = lax.iota(jnp.int32, L)
    ones = jnp.ones((L,), jnp.int32)
    zi16 = jnp.zeros((L,), jnp.int32)
    zf = jnp.zeros((L,), jnp.float32)

    wid = lax.axis_index("s") * 2 + lax.axis_index("c")
    base = wid * ROWS_PER_W

    def popcnt(m):
        return plsc.all_reduce_population_count(m)[0]

    def reduce_and_clear_hist():
        def rc(c, csum):
            tot_c = zi16
            for j in range(L):
                bv = hist[pl.ds((c * L + j) * L, L)]
                hist[pl.ds((c * L + j) * L, L)] = zi16
                tot_c = jnp.where(lanes == j, jnp.sum(bv), tot_c)
            tot[pl.ds(c * L, L)] = tot_c
            return jnp.where(lanes == c, jnp.sum(tot_c), csum)
        return lax.fori_loop(0, 16, rc, zi16)

    def boundary_find(csum, need):
        sfx_c = lax.rev(jnp.cumsum(lax.rev(csum, (0,))), (0,))
        cstar = jnp.sum((sfx_c >= need).astype(jnp.int32)) - 1
        prefix_above = jnp.sum(jnp.where(lanes == cstar, sfx_c - csum, 0))
        t_chunk = tot[pl.ds(cstar * L, L)]
        sfx2 = lax.rev(jnp.cumsum(lax.rev(t_chunk, (0,))), (0,)) + prefix_above
        l1 = jnp.sum((sfx2 >= need).astype(jnp.int32)) - 1
        b = cstar * L + l1
        g = jnp.sum(jnp.where(lanes == l1, sfx2 - t_chunk, 0))
        return b, need - g

    def work(row, hrow, crow, selref, sem, not_first):
        def h1(v, _):
            for u in range(UN):
                i = v * UN + u
                s = _sortable(hrow[pl.ds(i * L, L)])
                bkt = lax.shift_right_arithmetic(s, 24) + 128
                plsc.addupdate_scatter(hist, [bkt * L + lanes], ones)
            return 0
        lax.fori_loop(0, NVREG // UN, h1, 0)
        b1, need = boundary_find(reduce_and_clear_hist(), K)

        def h2(v, pos):
            for u in range(UN):
                i = v * UN + u
                s = _sortable(hrow[pl.ds(i * L, L)])
                bkt = lax.shift_right_arithmetic(s, 24) + 128
                m = bkt >= b1
                plsc.store_compressed(cand_s.at[pl.ds(pos, L)], s, mask=m)
                plsc.store_compressed(
                    cand_i.at[pl.ds(pos, L)], i * L + lanes, mask=m)
                meq = bkt == b1
                b2v = lax.shift_right_arithmetic(s, 16) & 0xFF
                plsc.addupdate_scatter(
                    hist, [b2v * L + lanes], ones, mask=meq)
                pos = pos + popcnt(m)
            return pos
        cnt = lax.fori_loop(0, NVREG // UN, h2, jnp.int32(0))
        b2, need = boundary_find(reduce_and_clear_hist(), need)
        p16 = (b1 - 128) * 256 + b2

        nv = lax.div(cnt + (L - 1), L)

        def h3(v, _):
            sv = cand_s[pl.ds(v * L, L)]
            valid = (v * L + lanes) < cnt
            m = valid & (lax.shift_right_arithmetic(sv, 16) == p16)
            b3v = lax.shift_right_arithmetic(sv, 8) & 0xFF
            plsc.addupdate_scatter(hist, [b3v * L + lanes], ones, mask=m)
            return 0
        lax.fori_loop(0, nv, h3, 0)
        b3, need = boundary_find(reduce_and_clear_hist(), need)
        p24 = p16 * 256 + b3

        def h4(v, _):
            sv = cand_s[pl.ds(v * L, L)]
            valid = (v * L + lanes) < cnt
            m = valid & (lax.shift_right_arithmetic(sv, 8) == p24)
            plsc.addupdate_scatter(hist, [(sv & 0xFF) * L + lanes], ones,
                                   mask=m)
            return 0
        lax.fori_loop(0, nv, h4, 0)
        b4, need_final = boundary_find(reduce_and_clear_hist(), need)
        t_s = p24 * 256 + b4

        @pl.when(not_first)
        def _():
            pltpu.make_async_copy(crow, codes_hbm.at[row - 2], sem).wait()
            for j in range(K // L):
                zi = selref[pl.ds(j * L, L)]
                plsc.store_scatter(crow, [zi], zf)

        def ap(v, carry):
            pos2, eqc = carry
            sv = cand_s[pl.ds(v * L, L)]
            iv = cand_i[pl.ds(v * L, L)]
            valid = (v * L + lanes) < cnt
            gt = valid & (sv > t_s)
            eq = valid & (sv == t_s)
            eqi = eq.astype(jnp.int32)
            rank = eqc + jnp.cumsum(eqi) - eqi
            take = gt | (eq & (rank < need_final))
            fv = lax.bitcast_convert_type(
                sv ^ (lax.shift_right_arithmetic(sv, 31)
                      & jnp.int32(0x7FFFFFFF)),
                jnp.float32)
            val = jnp.maximum(fv, 0.0)
            plsc.store_scatter(crow, [iv], val, mask=take)
            plsc.store_compressed(selref.at[pl.ds(pos2, L)], iv, mask=take)
            return (pos2 + popcnt(take), eqc + popcnt(eq))
        lax.fori_loop(0, nv, ap, (jnp.int32(0), jnp.int32(0)))
        pltpu.async_copy(crow, codes_hbm.at[row], sem)

    def zb(i, _):
        crow_a[pl.ds(i * L, L)] = zf
        crow_b[pl.ds(i * L, L)] = zf
        return 0
    lax.fori_loop(0, NVREG, zb, 0)

    def zh(i, _):
        hist[pl.ds(i * L, L)] = zi16
        return 0
    lax.fori_loop(0, 256, zh, 0)

    pltpu.async_copy(h_hbm.at[base], hrow_a, sem_ha)

    def pair_body(i, _):
        row0 = base + 2 * i
        pltpu.async_copy(h_hbm.at[row0 + 1], hrow_b, sem_hb)
        pltpu.make_async_copy(h_hbm.at[row0], hrow_a, sem_ha).wait()
        work(row0, hrow_a, crow_a, sel_a, sem_a, i >= 1)

        @pl.when(i < ROWS_PER_W // 2 - 1)
        def _():
            pltpu.async_copy(h_hbm.at[row0 + 2], hrow_a, sem_ha)

        pltpu.make_async_copy(h_hbm.at[row0 + 1], hrow_b, sem_hb).wait()
        work(row0 + 1, hrow_b, crow_b, sel_b, sem_b, i >= 1)
        return 0
    lax.fori_loop(0, ROWS_PER_W // 2, pair_body, 0)

    pltpu.make_async_copy(
        crow_a, codes_hbm.at[base + ROWS_PER_W - 2], sem_a).wait()
    pltpu.make_async_copy(
        crow_b, codes_hbm.at[base + ROWS_PER_W - 1], sem_b).wait()


def _sc_topk_codes(h):
    mesh = plsc.VectorSubcoreMesh(
        core_axis_name="c", subcore_axis_name="s", num_cores=2)
    f = pl.kernel(
        _sc_body,
        out_type=jax.ShapeDtypeStruct((NTOK, WIDTH), jnp.float32),
        mesh=mesh,
        compiler_params=pltpu.CompilerParams(needs_layout_passes=False),
        scratch_types=[
            pltpu.VMEM((WIDTH,), jnp.float32),
            pltpu.VMEM((WIDTH,), jnp.float32),
            pltpu.VMEM((WIDTH + L,), jnp.int32),
            pltpu.VMEM((WIDTH + L,), jnp.int32),
            pltpu.VMEM((256 * L,), jnp.int32),
            pltpu.VMEM((256,), jnp.int32),
            pltpu.VMEM((K + L,), jnp.int32),
            pltpu.VMEM((K + L,), jnp.int32),
            pltpu.VMEM((WIDTH,), jnp.float32),
            pltpu.VMEM((WIDTH,), jnp.float32),
            pltpu.SemaphoreType.DMA,
            pltpu.SemaphoreType.DMA,
            pltpu.SemaphoreType.DMA,
            pltpu.SemaphoreType.DMA,
        ],
    )
    return f(h)


@jax.jit
def kernel(x, Ae, be, bd, Ad):
    n = x.shape[0]
    xc = x - bd

    bm, bn = 256, 1024
    h = pl.pallas_call(
        _enc_body,
        grid=(n // bm, WIDTH // bn),
        in_specs=[
            pl.BlockSpec((bm, INPUT_DIM), lambda i, j: (i, 0)),
            pl.BlockSpec((bn, INPUT_DIM), lambda i, j: (j, 0)),
            pl.BlockSpec((1, bn), lambda i, j: (0, j)),
        ],
        out_specs=pl.BlockSpec((bm, bn), lambda i, j: (i, j)),
        out_shape=jax.ShapeDtypeStruct((n, WIDTH), jnp.float32),
    )(xc, Ae, be)

    codes = _sc_topk_codes(h)

    bm2, bn2, bk2 = 512, 1024, 2048
    out = pl.pallas_call(
        _dec_body,
        grid=(n // bm2, INPUT_DIM // bn2, WIDTH // bk2),
        in_specs=[
            pl.BlockSpec((bm2, bk2), lambda i, j, k: (i, k)),
            pl.BlockSpec((bn2, bk2), lambda i, j, k: (j, k)),
            pl.BlockSpec((1, bn2), lambda i, j, k: (0, j)),
        ],
        out_specs=pl.BlockSpec((bm2, bn2), lambda i, j, k: (i, j)),
        out_shape=jax.ShapeDtypeStruct((n, INPUT_DIM), jnp.float32),
        compiler_params=pltpu.CompilerParams(
            dimension_semantics=("parallel", "parallel", "arbitrary"),
        ),
    )(codes, Ad, bd)
    return out


# speedup vs baseline: 9.4394x; 2.2207x over previous
import functools

import jax
import jax.numpy as jnp
from jax import lax
from jax.experimental import pallas as pl
from jax.experimental.pallas import tpu as pltpu
from jax.experimental.pallas import tpu_sc as plsc

INPUT_DIM = 2048
WIDTH = 16384
NTOK = 2048
K = 64
L = 16
NW = 32
ROWS_PER_W = NTOK // NW
NVREG = WIDTH // L
UN = 8

_CONTRACT_LAST = (((1,), (1,)), ((), ()))


def _enc_body(x_ref, ae_ref, be_ref, h_ref):
    h_ref[...] = (
        lax.dot_general(x_ref[...], ae_ref[...], _CONTRACT_LAST,
                        preferred_element_type=jnp.float32)
        + be_ref[...]
    )


def _dec_body(c_ref, ad_ref, bd_ref, o_ref):
    k = pl.program_id(2)

    @pl.when(k == 0)
    def _():
        o_ref[...] = jnp.broadcast_to(bd_ref[...], o_ref.shape)

    o_ref[...] += lax.dot_general(c_ref[...], ad_ref[...], _CONTRACT_LAST,
                                  preferred_element_type=jnp.float32)


def _sortable(f):
    b = lax.bitcast_convert_type(f, jnp.int32)
    return b ^ (lax.shift_right_arithmetic(b, 31) & jnp.int32(0x7FFFFFFF))


def _sc_body(h_hbm, codes_hbm, hrow_a, hrow_b, cand_s, cand_i, hist, tot,
             sel_a, sel_b, crow_a, crow_b, sem_a, sem_b, sem_ha, sem_hb):
    lanes = lax.iota(jnp.int32, L)
    ones = jnp.ones((L,), jnp.int32)
    zi16 = jnp.zeros((L,), jnp.int32)
    zf = jnp.zeros((L,), jnp.float32)

    wid = lax.axis_index("s") * 2 + lax.axis_index("c")
    base = wid * ROWS_PER_W

    def popcnt(m):
        return plsc.all_reduce_population_count(m)[0]

    def reduce_and_clear_hist():
        def rc(c, csum):
            tot_c = zi16
            for j in range(L):
                bv = hist[pl.ds((c * L + j) * L, L)]
                hist[pl.ds((c * L + j) * L, L)] = zi16
                tot_c = jnp.where(lanes == j, jnp.sum(bv), tot_c)
            tot[pl.ds(c * L, L)] = tot_c
            return jnp.where(lanes == c, jnp.sum(tot_c), csum)
        return lax.fori_loop(0, 16, rc, zi16)

    def boundary_find(csum, need):
        sfx_c = lax.rev(jnp.cumsum(lax.rev(csum, (0,))), (0,))
        cstar = jnp.sum((sfx_c >= need).astype(jnp.int32)) - 1
        prefix_above = jnp.sum(jnp.where(lanes == cstar, sfx_c - csum, 0))
        t_chunk = tot[pl.ds(cstar * L, L)]
        sfx2 = lax.rev(jnp.cumsum(lax.rev(t_chunk, (0,))), (0,)) + prefix_above
        l1 = jnp.sum((sfx2 >= need).astype(jnp.int32)) - 1
        b = cstar * L + l1
        g = jnp.sum(jnp.where(lanes == l1, sfx2 - t_chunk, 0))
        return b, need - g

    def work(row, hrow, crow, selref, sem, not_first):
        @plsc.parallel_loop(0, NVREG, unroll=UN)
        def _(i):
            s = _sortable(hrow[pl.ds(i * L, L)])
            bkt = lax.shift_right_arithmetic(s, 24) + 128
            plsc.addupdate_scatter(hist, [bkt * L + lanes], ones)
        b1, need = boundary_find(reduce_and_clear_hist(), K)

        @plsc.parallel_loop(0, NVREG, unroll=UN, carry=jnp.int32(0))
        def cnt(i, pos):
            s = _sortable(hrow[pl.ds(i * L, L)])
            bkt = lax.shift_right_arithmetic(s, 24) + 128
            m = bkt >= b1
            plsc.store_compressed(cand_s.at[pl.ds(pos, L)], s, mask=m)
            plsc.store_compressed(
                cand_i.at[pl.ds(pos, L)], i * L + lanes, mask=m)
            meq = bkt == b1
            b2v = lax.shift_right_arithmetic(s, 16) & 0xFF
            plsc.addupdate_scatter(hist, [b2v * L + lanes], ones, mask=meq)
            return pos + popcnt(m)
        b2, need = boundary_find(reduce_and_clear_hist(), need)
        p16 = (b1 - 128) * 256 + b2

        nv = lax.div(cnt + (L - 1), L)

        @plsc.parallel_loop(0, nv, unroll=2)
        def _(v):
            sv = cand_s[pl.ds(v * L, L)]
            valid = (v * L + lanes) < cnt
            m = valid & (lax.shift_right_arithmetic(sv, 16) == p16)
            b3v = lax.shift_right_arithmetic(sv, 8) & 0xFF
            plsc.addupdate_scatter(hist, [b3v * L + lanes], ones, mask=m)
        b3, need = boundary_find(reduce_and_clear_hist(), need)
        p24 = p16 * 256 + b3

        @plsc.parallel_loop(0, nv, unroll=2)
        def _(v):
            sv = cand_s[pl.ds(v * L, L)]
            valid = (v * L + lanes) < cnt
            m = valid & (lax.shift_right_arithmetic(sv, 8) == p24)
            plsc.addupdate_scatter(hist, [(sv & 0xFF) * L + lanes], ones,
                                   mask=m)
        b4, need_final = boundary_find(reduce_and_clear_hist(), need)
        t_s = p24 * 256 + b4

        @pl.when(not_first)
        def _():
            pltpu.make_async_copy(crow, codes_hbm.at[row - 2], sem).wait()
            for j in range(K // L):
                zi = selref[pl.ds(j * L, L)]
                plsc.store_scatter(crow, [zi], zf)

        @plsc.parallel_loop(0, nv, unroll=2,
                            carry=(jnp.int32(0), jnp.int32(0)))
        def _(v, carry):
            pos2, eqc = carry
            sv = cand_s[pl.ds(v * L, L)]
            iv = cand_i[pl.ds(v * L, L)]
            valid = (v * L + lanes) < cnt
            gt = valid & (sv > t_s)
            eq = valid & (sv == t_s)
            eqi = eq.astype(jnp.int32)
            rank = eqc + jnp.cumsum(eqi) - eqi
            take = gt | (eq & (rank < need_final))
            fv = lax.bitcast_convert_type(
                sv ^ (lax.shift_right_arithmetic(sv, 31)
                      & jnp.int32(0x7FFFFFFF)),
                jnp.float32)
            val = jnp.maximum(fv, 0.0)
            plsc.store_scatter(crow, [iv], val, mask=take)
            plsc.store_compressed(selref.at[pl.ds(pos2, L)], iv, mask=take)
            return (pos2 + popcnt(take), eqc + popcnt(eq))
        pltpu.async_copy(crow, codes_hbm.at[row], sem)

    @plsc.parallel_loop(0, NVREG, unroll=UN)
    def _(i):
        crow_a[pl.ds(i * L, L)] = zf
        crow_b[pl.ds(i * L, L)] = zf

    @plsc.parallel_loop(0, 256, unroll=UN)
    def _(i):
        hist[pl.ds(i * L, L)] = zi16

    pltpu.async_copy(h_hbm.at[base], hrow_a, sem_ha)

    def pair_body(i, _):
        row0 = base + 2 * i
        pltpu.async_copy(h_hbm.at[row0 + 1], hrow_b, sem_hb)
        pltpu.make_async_copy(h_hbm.at[row0], hrow_a, sem_ha).wait()
        work(row0, hrow_a, crow_a, sel_a, sem_a, i >= 1)

        @pl.when(i < ROWS_PER_W // 2 - 1)
        def _():
            pltpu.async_copy(h_hbm.at[row0 + 2], hrow_a, sem_ha)

        pltpu.make_async_copy(h_hbm.at[row0 + 1], hrow_b, sem_hb).wait()
        work(row0 + 1, hrow_b, crow_b, sel_b, sem_b, i >= 1)
        return 0
    lax.fori_loop(0, ROWS_PER_W // 2, pair_body, 0)

    pltpu.make_async_copy(
        crow_a, codes_hbm.at[base + ROWS_PER_W - 2], sem_a).wait()
    pltpu.make_async_copy(
        crow_b, codes_hbm.at[base + ROWS_PER_W - 1], sem_b).wait()


def _sc_topk_codes(h):
    mesh = plsc.VectorSubcoreMesh(
        core_axis_name="c", subcore_axis_name="s", num_cores=2)
    f = pl.kernel(
        _sc_body,
        out_type=jax.ShapeDtypeStruct((NTOK, WIDTH), jnp.float32),
        mesh=mesh,
        compiler_params=pltpu.CompilerParams(needs_layout_passes=False),
        scratch_types=[
            pltpu.VMEM((WIDTH,), jnp.float32),
            pltpu.VMEM((WIDTH,), jnp.float32),
            pltpu.VMEM((WIDTH + L,), jnp.int32),
            pltpu.VMEM((WIDTH + L,), jnp.int32),
            pltpu.VMEM((256 * L,), jnp.int32),
            pltpu.VMEM((256,), jnp.int32),
            pltpu.VMEM((K + L,), jnp.int32),
            pltpu.VMEM((K + L,), jnp.int32),
            pltpu.VMEM((WIDTH,), jnp.float32),
            pltpu.VMEM((WIDTH,), jnp.float32),
            pltpu.SemaphoreType.DMA,
            pltpu.SemaphoreType.DMA,
            pltpu.SemaphoreType.DMA,
            pltpu.SemaphoreType.DMA,
        ],
    )
    return f(h)


@jax.jit
def kernel(x, Ae, be, bd, Ad):
    n = x.shape[0]
    xc = x - bd

    bm, bn = 256, 1024
    h = pl.pallas_call(
        _enc_body,
        grid=(n // bm, WIDTH // bn),
        in_specs=[
            pl.BlockSpec((bm, INPUT_DIM), lambda i, j: (i, 0)),
            pl.BlockSpec((bn, INPUT_DIM), lambda i, j: (j, 0)),
            pl.BlockSpec((1, bn), lambda i, j: (0, j)),
        ],
        out_specs=pl.BlockSpec((bm, bn), lambda i, j: (i, j)),
        out_shape=jax.ShapeDtypeStruct((n, WIDTH), jnp.float32),
    )(xc, Ae, be)

    codes = _sc_topk_codes(h)

    bm2, bn2, bk2 = 512, 1024, 2048
    out = pl.pallas_call(
        _dec_body,
        grid=(n // bm2, INPUT_DIM // bn2, WIDTH // bk2),
        in_specs=[
            pl.BlockSpec((bm2, bk2), lambda i, j, k: (i, k)),
            pl.BlockSpec((bn2, bk2), lambda i, j, k: (j, k)),
            pl.BlockSpec((1, bn2), lambda i, j, k: (0, j)),
        ],
        out_specs=pl.BlockSpec((bm2, bn2), lambda i, j, k: (i, j)),
        out_shape=jax.ShapeDtypeStruct((n, INPUT_DIM), jnp.float32),
        compiler_params=pltpu.CompilerParams(
            dimension_semantics=("parallel", "parallel", "arbitrary"),
        ),
    )(codes, Ad, bd)
    return out


# speedup vs baseline: 9.5927x; 1.0162x over previous
import functools

import jax
import jax.numpy as jnp
from jax import lax
from jax.experimental import pallas as pl
from jax.experimental.pallas import tpu as pltpu
from jax.experimental.pallas import tpu_sc as plsc

INPUT_DIM = 2048
WIDTH = 16384
NTOK = 2048
K = 64
L = 16
NW = 32
ROWS_PER_W = NTOK // NW
NVREG = WIDTH // L
UN = 8

_CONTRACT_LAST = (((1,), (1,)), ((), ()))


def _enc_body(x_ref, ae_ref, be_ref, h_ref):
    h_ref[...] = (
        lax.dot_general(x_ref[...], ae_ref[...], _CONTRACT_LAST,
                        preferred_element_type=jnp.float32)
        + be_ref[...]
    )


def _dec_body(c_ref, ad_ref, bd_ref, o_ref):
    k = pl.program_id(2)

    @pl.when(k == 0)
    def _():
        o_ref[...] = jnp.broadcast_to(bd_ref[...], o_ref.shape)

    o_ref[...] += lax.dot_general(c_ref[...], ad_ref[...], _CONTRACT_LAST,
                                  preferred_element_type=jnp.float32)


def _sortable(f):
    b = lax.bitcast_convert_type(f, jnp.int32)
    return b ^ (lax.shift_right_arithmetic(b, 31) & jnp.int32(0x7FFFFFFF))


def _sc_body(h_hbm, codes_hbm, hrow_a, hrow_b, cand_s, cand_i, hist, tot,
             sel_a, sel_b, crow_a, crow_b, sem_a, sem_b, sem_ha, sem_hb):
    lanes = lax.iota(jnp.int32, L)
    ones = jnp.ones((L,), jnp.int32)
    zi16 = jnp.zeros((L,), jnp.int32)
    zf = jnp.zeros((L,), jnp.float32)

    wid = lax.axis_index("s") * 2 + lax.axis_index("c")
    base = wid * ROWS_PER_W

    def popcnt(m):
        return plsc.all_reduce_population_count(m)[0]

    def reduce_and_clear_hist():
        @plsc.parallel_loop(0, 16, unroll=2, carry=zi16)
        def csum(c, csum):
            tot_c = zi16
            for j in range(L):
                bv = hist[pl.ds((c * L + j) * L, L)]
                hist[pl.ds((c * L + j) * L, L)] = zi16
                tot_c = jnp.where(lanes == j, jnp.sum(bv), tot_c)
            tot[pl.ds(c * L, L)] = tot_c
            return jnp.where(lanes == c, jnp.sum(tot_c), csum)
        return csum

    def boundary_find(csum, need):
        sfx_c = lax.rev(jnp.cumsum(lax.rev(csum, (0,))), (0,))
        cstar = jnp.sum((sfx_c >= need).astype(jnp.int32)) - 1
        prefix_above = jnp.sum(jnp.where(lanes == cstar, sfx_c - csum, 0))
        t_chunk = tot[pl.ds(cstar * L, L)]
        sfx2 = lax.rev(jnp.cumsum(lax.rev(t_chunk, (0,))), (0,)) + prefix_above
        l1 = jnp.sum((sfx2 >= need).astype(jnp.int32)) - 1
        b = cstar * L + l1
        g = jnp.sum(jnp.where(lanes == l1, sfx2 - t_chunk, 0))
        return b, need - g

    def work(row, hrow, crow, selref, sem, not_first):
        @plsc.parallel_loop(0, NVREG, unroll=UN)
        def _(i):
            s = _sortable(hrow[pl.ds(i * L, L)])
            bkt = lax.shift_right_arithmetic(s, 24) + 128
            plsc.addupdate_scatter(hist, [bkt * L + lanes], ones)
        b1, need = boundary_find(reduce_and_clear_hist(), K)

        @plsc.parallel_loop(0, NVREG, unroll=UN, carry=jnp.int32(0))
        def cnt(i, pos):
            s = _sortable(hrow[pl.ds(i * L, L)])
            bkt = lax.shift_right_arithmetic(s, 24) + 128
            m = bkt >= b1
            plsc.store_compressed(cand_s.at[pl.ds(pos, L)], s, mask=m)
            plsc.store_compressed(
                cand_i.at[pl.ds(pos, L)], i * L + lanes, mask=m)
            meq = bkt == b1
            b2v = lax.shift_right_arithmetic(s, 16) & 0xFF
            plsc.addupdate_scatter(hist, [b2v * L + lanes], ones, mask=meq)
            return pos + popcnt(m)
        b2, need = boundary_find(reduce_and_clear_hist(), need)
        p16 = (b1 - 128) * 256 + b2

        nv = lax.div(cnt + (L - 1), L)

        @plsc.parallel_loop(0, nv, unroll=2)
        def _(v):
            sv = cand_s[pl.ds(v * L, L)]
            valid = (v * L + lanes) < cnt
            m = valid & (lax.shift_right_arithmetic(sv, 16) == p16)
            b3v = lax.shift_right_arithmetic(sv, 8) & 0xFF
            plsc.addupdate_scatter(hist, [b3v * L + lanes], ones, mask=m)
        b3, need = boundary_find(reduce_and_clear_hist(), need)
        p24 = p16 * 256 + b3

        @plsc.parallel_loop(0, nv, unroll=2)
        def _(v):
            sv = cand_s[pl.ds(v * L, L)]
            valid = (v * L + lanes) < cnt
            m = valid & (lax.shift_right_arithmetic(sv, 8) == p24)
            plsc.addupdate_scatter(hist, [(sv & 0xFF) * L + lanes], ones,
                                   mask=m)
        b4, need_final = boundary_find(reduce_and_clear_hist(), need)
        t_s = p24 * 256 + b4

        @pl.when(not_first)
        def _():
            pltpu.make_async_copy(crow, codes_hbm.at[row - 2], sem).wait()
            for j in range(K // L):
                zi = selref[pl.ds(j * L, L)]
                plsc.store_scatter(crow, [zi], zf)

        @plsc.parallel_loop(0, nv, unroll=2,
                            carry=(jnp.int32(0), jnp.int32(0)))
        def _(v, carry):
            pos2, eqc = carry
            sv = cand_s[pl.ds(v * L, L)]
            iv = cand_i[pl.ds(v * L, L)]
            valid = (v * L + lanes) < cnt
            gt = valid & (sv > t_s)
            eq = valid & (sv == t_s)
            eqi = eq.astype(jnp.int32)
            rank = eqc + jnp.cumsum(eqi) - eqi
            take = gt | (eq & (rank < need_final))
            fv = lax.bitcast_convert_type(
                sv ^ (lax.shift_right_arithmetic(sv, 31)
                      & jnp.int32(0x7FFFFFFF)),
                jnp.float32)
            val = jnp.maximum(fv, 0.0)
            plsc.store_scatter(crow, [iv], val, mask=take)
            plsc.store_compressed(selref.at[pl.ds(pos2, L)], iv, mask=take)
            return (pos2 + popcnt(take), eqc + popcnt(eq))
        pltpu.async_copy(crow, codes_hbm.at[row], sem)

    @plsc.parallel_loop(0, NVREG, unroll=UN)
    def _(i):
        crow_a[pl.ds(i * L, L)] = zf
        crow_b[pl.ds(i * L, L)] = zf

    @plsc.parallel_loop(0, 256, unroll=UN)
    def _(i):
        hist[pl.ds(i * L, L)] = zi16

    pltpu.async_copy(h_hbm.at[base], hrow_a, sem_ha)

    def pair_body(i, _):
        row0 = base + 2 * i
        pltpu.async_copy(h_hbm.at[row0 + 1], hrow_b, sem_hb)
        pltpu.make_async_copy(h_hbm.at[row0], hrow_a, sem_ha).wait()
        work(row0, hrow_a, crow_a, sel_a, sem_a, i >= 1)

        @pl.when(i < ROWS_PER_W // 2 - 1)
        def _():
            pltpu.async_copy(h_hbm.at[row0 + 2], hrow_a, sem_ha)

        pltpu.make_async_copy(h_hbm.at[row0 + 1], hrow_b, sem_hb).wait()
        work(row0 + 1, hrow_b, crow_b, sel_b, sem_b, i >= 1)
        return 0
    lax.fori_loop(0, ROWS_PER_W // 2, pair_body, 0)

    pltpu.make_async_copy(
        crow_a, codes_hbm.at[base + ROWS_PER_W - 2], sem_a).wait()
    pltpu.make_async_copy(
        crow_b, codes_hbm.at[base + ROWS_PER_W - 1], sem_b).wait()


def _sc_topk_codes(h):
    mesh = plsc.VectorSubcoreMesh(
        core_axis_name="c", subcore_axis_name="s", num_cores=2)
    f = pl.kernel(
        _sc_body,
        out_type=jax.ShapeDtypeStruct((NTOK, WIDTH), jnp.float32),
        mesh=mesh,
        compiler_params=pltpu.CompilerParams(needs_layout_passes=False),
        scratch_types=[
            pltpu.VMEM((WIDTH,), jnp.float32),
            pltpu.VMEM((WIDTH,), jnp.float32),
            pltpu.VMEM((WIDTH + L,), jnp.int32),
            pltpu.VMEM((WIDTH + L,), jnp.int32),
            pltpu.VMEM((256 * L,), jnp.int32),
            pltpu.VMEM((256,), jnp.int32),
            pltpu.VMEM((K + L,), jnp.int32),
            pltpu.VMEM((K + L,), jnp.int32),
            pltpu.VMEM((WIDTH,), jnp.float32),
            pltpu.VMEM((WIDTH,), jnp.float32),
            pltpu.SemaphoreType.DMA,
            pltpu.SemaphoreType.DMA,
            pltpu.SemaphoreType.DMA,
            pltpu.SemaphoreType.DMA,
        ],
    )
    return f(h)


@jax.jit
def kernel(x, Ae, be, bd, Ad):
    n = x.shape[0]
    xc = x - bd

    bm, bn = 256, 1024
    h = pl.pallas_call(
        _enc_body,
        grid=(n // bm, WIDTH // bn),
        in_specs=[
            pl.BlockSpec((bm, INPUT_DIM), lambda i, j: (i, 0)),
            pl.BlockSpec((bn, INPUT_DIM), lambda i, j: (j, 0)),
            pl.BlockSpec((1, bn), lambda i, j: (0, j)),
        ],
        out_specs=pl.BlockSpec((bm, bn), lambda i, j: (i, j)),
        out_shape=jax.ShapeDtypeStruct((n, WIDTH), jnp.float32),
    )(xc, Ae, be)

    codes = _sc_topk_codes(h)

    bm2, bn2, bk2 = 512, 1024, 2048
    out = pl.pallas_call(
        _dec_body,
        grid=(n // bm2, INPUT_DIM // bn2, WIDTH // bk2),
        in_specs=[
            pl.BlockSpec((bm2, bk2), lambda i, j, k: (i, k)),
            pl.BlockSpec((bn2, bk2), lambda i, j, k: (j, k)),
            pl.BlockSpec((1, bn2), lambda i, j, k: (0, j)),
        ],
        out_specs=pl.BlockSpec((bm2, bn2), lambda i, j, k: (i, j)),
        out_shape=jax.ShapeDtypeStruct((n, INPUT_DIM), jnp.float32),
        compiler_params=pltpu.CompilerParams(
            dimension_semantics=("parallel", "parallel", "arbitrary"),
        ),
    )(codes, Ad, bd)
    return out


# speedup vs baseline: 12.3874x; 1.2913x over previous
import functools

import jax
import jax.numpy as jnp
from jax import lax
from jax.experimental import pallas as pl
from jax.experimental.pallas import tpu as pltpu
from jax.experimental.pallas import tpu_sc as plsc

INPUT_DIM = 2048
WIDTH = 16384
NTOK = 2048
K = 64
L = 16
NW = 32
ROWS_PER_W = NTOK // NW
NVREG = WIDTH // L
UN = 8

_CONTRACT_LAST = (((1,), (1,)), ((), ()))


def _enc_body(x_ref, ae_ref, be_ref, h_ref):
    h_ref[...] = (
        lax.dot_general(x_ref[...], ae_ref[...], _CONTRACT_LAST,
                        preferred_element_type=jnp.float32)
        + be_ref[...]
    )


def _dec_body(c_ref, ad_ref, bd_ref, o_ref):
    k = pl.program_id(2)

    @pl.when(k == 0)
    def _():
        o_ref[...] = jnp.broadcast_to(bd_ref[...], o_ref.shape)

    o_ref[...] += lax.dot_general(c_ref[...], ad_ref[...], _CONTRACT_LAST,
                                  preferred_element_type=jnp.float32)


def _sortable(f):
    b = lax.bitcast_convert_type(f, jnp.int32)
    return b ^ (lax.shift_right_arithmetic(b, 31) & jnp.int32(0x7FFFFFFF))


def _sc_body(rows_per_w, h_hbm, codes_hbm, hrow_a, hrow_b, cand_s, cand_i,
             hist, tot, sel_a, sel_b, crow_a, crow_b, sem_a, sem_b,
             sem_ha, sem_hb):
    lanes = lax.iota(jnp.int32, L)
    ones = jnp.ones((L,), jnp.int32)
    zi16 = jnp.zeros((L,), jnp.int32)
    zf = jnp.zeros((L,), jnp.float32)

    wid = lax.axis_index("s") * 2 + lax.axis_index("c")
    base = wid * rows_per_w

    def popcnt(m):
        return plsc.all_reduce_population_count(m)[0]

    def reduce_and_clear_hist():
        def rc(c, csum):
            tot_c = zi16
            for j in range(L):
                bv = hist[pl.ds((c * L + j) * L, L)]
                hist[pl.ds((c * L + j) * L, L)] = zi16
                tot_c = jnp.where(lanes == j, jnp.sum(bv), tot_c)
            tot[pl.ds(c * L, L)] = tot_c
            return jnp.where(lanes == c, jnp.sum(tot_c), csum)
        return lax.fori_loop(0, 16, rc, zi16)

    def boundary_find(csum, need):
        sfx_c = lax.rev(jnp.cumsum(lax.rev(csum, (0,))), (0,))
        cstar = jnp.sum((sfx_c >= need).astype(jnp.int32)) - 1
        prefix_above = jnp.sum(jnp.where(lanes == cstar, sfx_c - csum, 0))
        t_chunk = tot[pl.ds(cstar * L, L)]
        sfx2 = lax.rev(jnp.cumsum(lax.rev(t_chunk, (0,))), (0,)) + prefix_above
        l1 = jnp.sum((sfx2 >= need).astype(jnp.int32)) - 1
        b = cstar * L + l1
        g = jnp.sum(jnp.where(lanes == l1, sfx2 - t_chunk, 0))
        return b, need - g

    def work(row, hrow, crow, selref, sem, not_first):
        @plsc.parallel_loop(0, NVREG, unroll=UN)
        def _(i):
            s = _sortable(hrow[pl.ds(i * L, L)])
            bkt = lax.shift_right_arithmetic(s, 24) + 128
            plsc.addupdate_scatter(hist, [bkt * L + lanes], ones)
        b1, need = boundary_find(reduce_and_clear_hist(), K)

        @plsc.parallel_loop(0, NVREG, unroll=UN, carry=jnp.int32(0))
        def cnt(i, pos):
            s = _sortable(hrow[pl.ds(i * L, L)])
            bkt = lax.shift_right_arithmetic(s, 24) + 128
            m = bkt >= b1
            plsc.store_compressed(cand_s.at[pl.ds(pos, L)], s, mask=m)
            plsc.store_compressed(
                cand_i.at[pl.ds(pos, L)], i * L + lanes, mask=m)
            meq = bkt == b1
            b2v = lax.shift_right_arithmetic(s, 16) & 0xFF
            plsc.addupdate_scatter(hist, [b2v * L + lanes], ones, mask=meq)
            return pos + popcnt(m)
        b2, need = boundary_find(reduce_and_clear_hist(), need)
        p16 = (b1 - 128) * 256 + b2

        nv = lax.div(cnt + (L - 1), L)

        @plsc.parallel_loop(0, nv, unroll=2)
        def _(v):
            sv = cand_s[pl.ds(v * L, L)]
            valid = (v * L + lanes) < cnt
            m = valid & (lax.shift_right_arithmetic(sv, 16) == p16)
            b3v = lax.shift_right_arithmetic(sv, 8) & 0xFF
            plsc.addupdate_scatter(hist, [b3v * L + lanes], ones, mask=m)
        b3, need = boundary_find(reduce_and_clear_hist(), need)
        p24 = p16 * 256 + b3

        @plsc.parallel_loop(0, nv, unroll=2)
        def _(v):
            sv = cand_s[pl.ds(v * L, L)]
            valid = (v * L + lanes) < cnt
            m = valid & (lax.shift_right_arithmetic(sv, 8) == p24)
            plsc.addupdate_scatter(hist, [(sv & 0xFF) * L + lanes], ones,
                                   mask=m)
        b4, need_final = boundary_find(reduce_and_clear_hist(), need)
        t_s = p24 * 256 + b4

        @pl.when(not_first)
        def _():
            pltpu.make_async_copy(crow, codes_hbm.at[row - 2], sem).wait()
            for j in range(K // L):
                zi = selref[pl.ds(j * L, L)]
                plsc.store_scatter(crow, [zi], zf)

        @plsc.parallel_loop(0, nv, unroll=2,
                            carry=(jnp.int32(0), jnp.int32(0)))
        def _(v, carry):
            pos2, eqc = carry
            sv = cand_s[pl.ds(v * L, L)]
            iv = cand_i[pl.ds(v * L, L)]
            valid = (v * L + lanes) < cnt
            gt = valid & (sv > t_s)
            eq = valid & (sv == t_s)
            eqi = eq.astype(jnp.int32)
            rank = eqc + jnp.cumsum(eqi) - eqi
            take = gt | (eq & (rank < need_final))
            fv = lax.bitcast_convert_type(
                sv ^ (lax.shift_right_arithmetic(sv, 31)
                      & jnp.int32(0x7FFFFFFF)),
                jnp.float32)
            val = jnp.maximum(fv, 0.0)
            plsc.store_scatter(crow, [iv], val, mask=take)
            plsc.store_compressed(selref.at[pl.ds(pos2, L)], iv, mask=take)
            return (pos2 + popcnt(take), eqc + popcnt(eq))
        pltpu.async_copy(crow, codes_hbm.at[row], sem)

    @plsc.parallel_loop(0, NVREG, unroll=UN)
    def _(i):
        crow_a[pl.ds(i * L, L)] = zf
        crow_b[pl.ds(i * L, L)] = zf

    @plsc.parallel_loop(0, 256, unroll=UN)
    def _(i):
        hist[pl.ds(i * L, L)] = zi16

    pltpu.async_copy(h_hbm.at[base], hrow_a, sem_ha)

    def pair_body(i, _):
        row0 = base + 2 * i
        pltpu.async_copy(h_hbm.at[row0 + 1], hrow_b, sem_hb)
        pltpu.make_async_copy(h_hbm.at[row0], hrow_a, sem_ha).wait()
        work(row0, hrow_a, crow_a, sel_a, sem_a, i >= 1)

        @pl.when(i < rows_per_w // 2 - 1)
        def _():
            pltpu.async_copy(h_hbm.at[row0 + 2], hrow_a, sem_ha)

        pltpu.make_async_copy(h_hbm.at[row0 + 1], hrow_b, sem_hb).wait()
        work(row0 + 1, hrow_b, crow_b, sel_b, sem_b, i >= 1)
        return 0
    lax.fori_loop(0, rows_per_w // 2, pair_body, 0)

    pltpu.make_async_copy(
        crow_a, codes_hbm.at[base + rows_per_w - 2], sem_a).wait()
    pltpu.make_async_copy(
        crow_b, codes_hbm.at[base + rows_per_w - 1], sem_b).wait()


def _sc_topk_codes(h):
    ntok = h.shape[0]
    mesh = plsc.VectorSubcoreMesh(
        core_axis_name="c", subcore_axis_name="s", num_cores=2)
    f = pl.kernel(
        functools.partial(_sc_body, ntok // NW),
        out_type=jax.ShapeDtypeStruct((ntok, WIDTH), jnp.float32),
        mesh=mesh,
        compiler_params=pltpu.CompilerParams(needs_layout_passes=False),
        scratch_types=[
            pltpu.VMEM((WIDTH,), jnp.float32),
            pltpu.VMEM((WIDTH,), jnp.float32),
            pltpu.VMEM((WIDTH + L,), jnp.int32),
            pltpu.VMEM((WIDTH + L,), jnp.int32),
            pltpu.VMEM((256 * L,), jnp.int32),
            pltpu.VMEM((256,), jnp.int32),
            pltpu.VMEM((K + L,), jnp.int32),
            pltpu.VMEM((K + L,), jnp.int32),
            pltpu.VMEM((WIDTH,), jnp.float32),
            pltpu.VMEM((WIDTH,), jnp.float32),
            pltpu.SemaphoreType.DMA,
            pltpu.SemaphoreType.DMA,
            pltpu.SemaphoreType.DMA,
            pltpu.SemaphoreType.DMA,
        ],
    )
    return f(h)


NCHUNK = 2


def _enc_chunk(xc, Ae, be, n):
    bm, bn = 256, 1024
    return pl.pallas_call(
        _enc_body,
        grid=(n // bm, WIDTH // bn),
        in_specs=[
            pl.BlockSpec((bm, INPUT_DIM), lambda i, j: (i, 0)),
            pl.BlockSpec((bn, INPUT_DIM), lambda i, j: (j, 0)),
            pl.BlockSpec((1, bn), lambda i, j: (0, j)),
        ],
        out_specs=pl.BlockSpec((bm, bn), lambda i, j: (i, j)),
        out_shape=jax.ShapeDtypeStruct((n, WIDTH), jnp.float32),
    )(xc, Ae, be)


def _dec_chunk(codes, Ad, bd, n):
    bm2, bn2, bk2 = 512, 1024, 2048
    return pl.pallas_call(
        _dec_body,
        grid=(n // bm2, INPUT_DIM // bn2, WIDTH // bk2),
        in_specs=[
            pl.BlockSpec((bm2, bk2), lambda i, j, k: (i, k)),
            pl.BlockSpec((bn2, bk2), lambda i, j, k: (j, k)),
            pl.BlockSpec((1, bn2), lambda i, j, k: (0, j)),
        ],
        out_specs=pl.BlockSpec((bm2, bn2), lambda i, j, k: (i, j)),
        out_shape=jax.ShapeDtypeStruct((n, INPUT_DIM), jnp.float32),
        compiler_params=pltpu.CompilerParams(
            dimension_semantics=("parallel", "parallel", "arbitrary"),
        ),
    )(codes, Ad, bd)


@jax.jit
def kernel(x, Ae, be, bd, Ad):
    n = x.shape[0]
    xc = x - bd
    nc = n // NCHUNK
    outs = []
    for ci in range(NCHUNK):
        xci = lax.slice_in_dim(xc, ci * nc, (ci + 1) * nc, axis=0)
        h = _enc_chunk(xci, Ae, be, nc)
        codes = _sc_topk_codes(h)
        outs.append(_dec_chunk(codes, Ad, bd, nc))
    return jnp.concatenate(outs, axis=0)


# speedup vs baseline: 13.7102x; 1.1068x over previous
import functools

import jax
import jax.numpy as jnp
from jax import lax
from jax.experimental import pallas as pl
from jax.experimental.pallas import tpu as pltpu
from jax.experimental.pallas import tpu_sc as plsc

INPUT_DIM = 2048
WIDTH = 16384
NTOK = 2048
K = 64
L = 16
NW = 32
ROWS_PER_W = NTOK // NW
NVREG = WIDTH // L
UN = 8

_CONTRACT_LAST = (((1,), (1,)), ((), ()))


def _enc_body(x_ref, ae_ref, be_ref, h_ref):
    h_ref[...] = (
        lax.dot_general(x_ref[...], ae_ref[...], _CONTRACT_LAST,
                        preferred_element_type=jnp.float32)
        + be_ref[...]
    )


def _dec_body(c_ref, ad_ref, bd_ref, o_ref):
    k = pl.program_id(2)

    @pl.when(k == 0)
    def _():
        o_ref[...] = jnp.broadcast_to(bd_ref[...], o_ref.shape)

    o_ref[...] += lax.dot_general(c_ref[...], ad_ref[...], _CONTRACT_LAST,
                                  preferred_element_type=jnp.float32)


def _sortable(f):
    b = lax.bitcast_convert_type(f, jnp.int32)
    return b ^ (lax.shift_right_arithmetic(b, 31) & jnp.int32(0x7FFFFFFF))


def _sc_body(rows_per_w, h_hbm, codes_hbm, hrow_a, hrow_b, cand_s, cand_i,
             hist, tot, sel_a, sel_b, crow_a, crow_b, sem_a, sem_b,
             sem_ha, sem_hb):
    lanes = lax.iota(jnp.int32, L)
    ones = jnp.ones((L,), jnp.int32)
    zi16 = jnp.zeros((L,), jnp.int32)
    zf = jnp.zeros((L,), jnp.float32)

    wid = lax.axis_index("s") * 2 + lax.axis_index("c")
    base = wid * rows_per_w

    def popcnt(m):
        return plsc.all_reduce_population_count(m)[0]

    def reduce_and_clear_hist():
        def rc(c, csum):
            tot_c = zi16
            for j in range(L):
                bv = hist[pl.ds((c * L + j) * L, L)]
                hist[pl.ds((c * L + j) * L, L)] = zi16
                tot_c = jnp.where(lanes == j, jnp.sum(bv), tot_c)
            tot[pl.ds(c * L, L)] = tot_c
            return jnp.where(lanes == c, jnp.sum(tot_c), csum)
        return lax.fori_loop(0, 16, rc, zi16)

    def boundary_find(csum, need):
        sfx_c = lax.rev(jnp.cumsum(lax.rev(csum, (0,))), (0,))
        cstar = jnp.sum((sfx_c >= need).astype(jnp.int32)) - 1
        prefix_above = jnp.sum(jnp.where(lanes == cstar, sfx_c - csum, 0))
        t_chunk = tot[pl.ds(cstar * L, L)]
        sfx2 = lax.rev(jnp.cumsum(lax.rev(t_chunk, (0,))), (0,)) + prefix_above
        l1 = jnp.sum((sfx2 >= need).astype(jnp.int32)) - 1
        b = cstar * L + l1
        g = jnp.sum(jnp.where(lanes == l1, sfx2 - t_chunk, 0))
        return b, need - g

    def work(row, hrow, crow, selref, sem, not_first):
        @plsc.parallel_loop(0, NVREG, unroll=UN)
        def _(i):
            s = _sortable(hrow[pl.ds(i * L, L)])
            bkt = lax.shift_right_arithmetic(s, 24) + 128
            plsc.addupdate_scatter(hist, [bkt * L + lanes], ones)
        b1, need = boundary_find(reduce_and_clear_hist(), K)

        @plsc.parallel_loop(0, NVREG, unroll=UN, carry=jnp.int32(0))
        def cnt(i, pos):
            s = _sortable(hrow[pl.ds(i * L, L)])
            bkt = lax.shift_right_arithmetic(s, 24) + 128
            m = bkt >= b1
            plsc.store_compressed(cand_s.at[pl.ds(pos, L)], s, mask=m)
            plsc.store_compressed(
                cand_i.at[pl.ds(pos, L)], i * L + lanes, mask=m)
            meq = bkt == b1
            b2v = lax.shift_right_arithmetic(s, 16) & 0xFF
            plsc.addupdate_scatter(hist, [b2v * L + lanes], ones, mask=meq)
            return pos + popcnt(m)
        b2, need = boundary_find(reduce_and_clear_hist(), need)
        p16 = (b1 - 128) * 256 + b2

        nv = lax.div(cnt + (L - 1), L)

        @plsc.parallel_loop(0, nv, unroll=2)
        def _(v):
            sv = cand_s[pl.ds(v * L, L)]
            valid = (v * L + lanes) < cnt
            m = valid & (lax.shift_right_arithmetic(sv, 16) == p16)
            b3v = lax.shift_right_arithmetic(sv, 8) & 0xFF
            plsc.addupdate_scatter(hist, [b3v * L + lanes], ones, mask=m)
        b3, need = boundary_find(reduce_and_clear_hist(), need)
        p24 = p16 * 256 + b3

        @plsc.parallel_loop(0, nv, unroll=2)
        def _(v):
            sv = cand_s[pl.ds(v * L, L)]
            valid = (v * L + lanes) < cnt
            m = valid & (lax.shift_right_arithmetic(sv, 8) == p24)
            plsc.addupdate_scatter(hist, [(sv & 0xFF) * L + lanes], ones,
                                   mask=m)
        b4, need_final = boundary_find(reduce_and_clear_hist(), need)
        t_s = p24 * 256 + b4

        @pl.when(not_first)
        def _():
            pltpu.make_async_copy(crow, codes_hbm.at[row - 2], sem).wait()
            for j in range(K // L):
                zi = selref[pl.ds(j * L, L)]
                plsc.store_scatter(crow, [zi], zf)

        @plsc.parallel_loop(0, nv, unroll=2,
                            carry=(jnp.int32(0), jnp.int32(0)))
        def _(v, carry):
            pos2, eqc = carry
            sv = cand_s[pl.ds(v * L, L)]
            iv = cand_i[pl.ds(v * L, L)]
            valid = (v * L + lanes) < cnt
            gt = valid & (sv > t_s)
            eq = valid & (sv == t_s)
            eqi = eq.astype(jnp.int32)
            rank = eqc + jnp.cumsum(eqi) - eqi
            take = gt | (eq & (rank < need_final))
            fv = lax.bitcast_convert_type(
                sv ^ (lax.shift_right_arithmetic(sv, 31)
                      & jnp.int32(0x7FFFFFFF)),
                jnp.float32)
            val = jnp.maximum(fv, 0.0)
            plsc.store_scatter(crow, [iv], val, mask=take)
            plsc.store_compressed(selref.at[pl.ds(pos2, L)], iv, mask=take)
            return (pos2 + popcnt(take), eqc + popcnt(eq))
        pltpu.async_copy(crow, codes_hbm.at[row], sem)

    @plsc.parallel_loop(0, NVREG, unroll=UN)
    def _(i):
        crow_a[pl.ds(i * L, L)] = zf
        crow_b[pl.ds(i * L, L)] = zf

    @plsc.parallel_loop(0, 256, unroll=UN)
    def _(i):
        hist[pl.ds(i * L, L)] = zi16

    pltpu.async_copy(h_hbm.at[base], hrow_a, sem_ha)

    def pair_body(i, _):
        row0 = base + 2 * i
        pltpu.async_copy(h_hbm.at[row0 + 1], hrow_b, sem_hb)
        pltpu.make_async_copy(h_hbm.at[row0], hrow_a, sem_ha).wait()
        work(row0, hrow_a, crow_a, sel_a, sem_a, i >= 1)

        @pl.when(i < rows_per_w // 2 - 1)
        def _():
            pltpu.async_copy(h_hbm.at[row0 + 2], hrow_a, sem_ha)

        pltpu.make_async_copy(h_hbm.at[row0 + 1], hrow_b, sem_hb).wait()
        work(row0 + 1, hrow_b, crow_b, sel_b, sem_b, i >= 1)
        return 0
    lax.fori_loop(0, rows_per_w // 2, pair_body, 0)

    pltpu.make_async_copy(
        crow_a, codes_hbm.at[base + rows_per_w - 2], sem_a).wait()
    pltpu.make_async_copy(
        crow_b, codes_hbm.at[base + rows_per_w - 1], sem_b).wait()


def _sc_topk_codes(h):
    ntok = h.shape[0]
    mesh = plsc.VectorSubcoreMesh(
        core_axis_name="c", subcore_axis_name="s", num_cores=2)
    f = pl.kernel(
        functools.partial(_sc_body, ntok // NW),
        out_type=jax.ShapeDtypeStruct((ntok, WIDTH), jnp.float32),
        mesh=mesh,
        compiler_params=pltpu.CompilerParams(needs_layout_passes=False),
        scratch_types=[
            pltpu.VMEM((WIDTH,), jnp.float32),
            pltpu.VMEM((WIDTH,), jnp.float32),
            pltpu.VMEM((WIDTH + L,), jnp.int32),
            pltpu.VMEM((WIDTH + L,), jnp.int32),
            pltpu.VMEM((256 * L,), jnp.int32),
            pltpu.VMEM((256,), jnp.int32),
            pltpu.VMEM((K + L,), jnp.int32),
            pltpu.VMEM((K + L,), jnp.int32),
            pltpu.VMEM((WIDTH,), jnp.float32),
            pltpu.VMEM((WIDTH,), jnp.float32),
            pltpu.SemaphoreType.DMA,
            pltpu.SemaphoreType.DMA,
            pltpu.SemaphoreType.DMA,
            pltpu.SemaphoreType.DMA,
        ],
    )
    return f(h)


NCHUNK = 4


def _enc_chunk(xc, Ae, be, n):
    bm, bn = 256, 1024
    return pl.pallas_call(
        _enc_body,
        grid=(n // bm, WIDTH // bn),
        in_specs=[
            pl.BlockSpec((bm, INPUT_DIM), lambda i, j: (i, 0)),
            pl.BlockSpec((bn, INPUT_DIM), lambda i, j: (j, 0)),
            pl.BlockSpec((1, bn), lambda i, j: (0, j)),
        ],
        out_specs=pl.BlockSpec((bm, bn), lambda i, j: (i, j)),
        out_shape=jax.ShapeDtypeStruct((n, WIDTH), jnp.float32),
    )(xc, Ae, be)


def _dec_chunk(codes, Ad, bd, n):
    bm2, bn2, bk2 = 512, 1024, 2048
    return pl.pallas_call(
        _dec_body,
        grid=(n // bm2, INPUT_DIM // bn2, WIDTH // bk2),
        in_specs=[
            pl.BlockSpec((bm2, bk2), lambda i, j, k: (i, k)),
            pl.BlockSpec((bn2, bk2), lambda i, j, k: (j, k)),
            pl.BlockSpec((1, bn2), lambda i, j, k: (0, j)),
        ],
        out_specs=pl.BlockSpec((bm2, bn2), lambda i, j, k: (i, j)),
        out_shape=jax.ShapeDtypeStruct((n, INPUT_DIM), jnp.float32),
        compiler_params=pltpu.CompilerParams(
            dimension_semantics=("parallel", "parallel", "arbitrary"),
        ),
    )(codes, Ad, bd)


@jax.jit
def kernel(x, Ae, be, bd, Ad):
    n = x.shape[0]
    xc = x - bd
    nc = n // NCHUNK
    outs = []
    for ci in range(NCHUNK):
        xci = lax.slice_in_dim(xc, ci * nc, (ci + 1) * nc, axis=0)
        h = _enc_chunk(xci, Ae, be, nc)
        codes = _sc_topk_codes(h)
        outs.append(_dec_chunk(codes, Ad, bd, nc))
    return jnp.concatenate(outs, axis=0)


# speedup vs baseline: 15.3627x; 1.1205x over previous
import functools

import jax
import jax.numpy as jnp
from jax import lax
from jax.experimental import pallas as pl
from jax.experimental.pallas import tpu as pltpu
from jax.experimental.pallas import tpu_sc as plsc

INPUT_DIM = 2048
WIDTH = 16384
NTOK = 2048
K = 64
L = 16
NW = 32
ROWS_PER_W = NTOK // NW
NVREG = WIDTH // L
UN = 8

_CONTRACT_LAST = (((1,), (1,)), ((), ()))


def _enc_body(x_ref, ae_ref, be_ref, h_ref):
    h_ref[...] = (
        lax.dot_general(x_ref[...], ae_ref[...], _CONTRACT_LAST,
                        preferred_element_type=jnp.float32)
        + be_ref[...]
    )


def _dec_body(c_ref, ad_ref, bd_ref, o_ref):
    k = pl.program_id(2)

    @pl.when(k == 0)
    def _():
        o_ref[...] = jnp.broadcast_to(bd_ref[...], o_ref.shape)

    o_ref[...] += lax.dot_general(c_ref[...], ad_ref[...], _CONTRACT_LAST,
                                  preferred_element_type=jnp.float32)


def _sortable(f):
    b = lax.bitcast_convert_type(f, jnp.int32)
    return b ^ (lax.shift_right_arithmetic(b, 31) & jnp.int32(0x7FFFFFFF))


def _sc_body(rows_per_w, h_hbm, codes_hbm, hrow_a, hrow_b, cand_s, cand_i,
             hist, tot, sel_a, sel_b, crow_a, crow_b, sem_a, sem_b,
             sem_ha, sem_hb):
    lanes = lax.iota(jnp.int32, L)
    ones = jnp.ones((L,), jnp.int32)
    zi16 = jnp.zeros((L,), jnp.int32)
    zf = jnp.zeros((L,), jnp.float32)

    wid = lax.axis_index("s") * 2 + lax.axis_index("c")
    base = wid * rows_per_w

    def popcnt(m):
        return plsc.all_reduce_population_count(m)[0]

    def reduce_and_clear_hist():
        def rc(c, csum):
            tot_c = zi16
            for j in range(L):
                bv = hist[pl.ds((c * L + j) * L, L)]
                hist[pl.ds((c * L + j) * L, L)] = zi16
                tot_c = jnp.where(lanes == j, jnp.sum(bv), tot_c)
            tot[pl.ds(c * L, L)] = tot_c
            return jnp.where(lanes == c, jnp.sum(tot_c), csum)
        return lax.fori_loop(0, 16, rc, zi16)

    def boundary_find(csum, need):
        sfx_c = lax.rev(jnp.cumsum(lax.rev(csum, (0,))), (0,))
        cstar = jnp.sum((sfx_c >= need).astype(jnp.int32)) - 1
        prefix_above = jnp.sum(jnp.where(lanes == cstar, sfx_c - csum, 0))
        t_chunk = tot[pl.ds(cstar * L, L)]
        sfx2 = lax.rev(jnp.cumsum(lax.rev(t_chunk, (0,))), (0,)) + prefix_above
        l1 = jnp.sum((sfx2 >= need).astype(jnp.int32)) - 1
        b = cstar * L + l1
        g = jnp.sum(jnp.where(lanes == l1, sfx2 - t_chunk, 0))
        return b, need - g

    def work(row, hrow, crow, selref, sem, not_first):
        @plsc.parallel_loop(0, NVREG, unroll=UN)
        def _(i):
            s = _sortable(hrow[pl.ds(i * L, L)])
            bkt = lax.shift_right_arithmetic(s, 24) + 128
            plsc.addupdate_scatter(hist, [bkt * L + lanes], ones)
        b1, need = boundary_find(reduce_and_clear_hist(), K)

        @plsc.parallel_loop(0, NVREG, unroll=UN, carry=jnp.int32(0))
        def cnt(i, pos):
            s = _sortable(hrow[pl.ds(i * L, L)])
            bkt = lax.shift_right_arithmetic(s, 24) + 128
            m = bkt >= b1
            plsc.store_compressed(cand_s.at[pl.ds(pos, L)], s, mask=m)
            plsc.store_compressed(
                cand_i.at[pl.ds(pos, L)], i * L + lanes, mask=m)
            meq = bkt == b1
            b2v = lax.shift_right_arithmetic(s, 16) & 0xFF
            plsc.addupdate_scatter(hist, [b2v * L + lanes], ones, mask=meq)
            return pos + popcnt(m)
        b2, need = boundary_find(reduce_and_clear_hist(), need)
        p16 = (b1 - 128) * 256 + b2

        nv = lax.div(cnt + (L - 1), L)

        @plsc.parallel_loop(0, nv, unroll=2)
        def _(v):
            sv = cand_s[pl.ds(v * L, L)]
            valid = (v * L + lanes) < cnt
            m = valid & (lax.shift_right_arithmetic(sv, 16) == p16)
            b3v = lax.shift_right_arithmetic(sv, 8) & 0xFF
            plsc.addupdate_scatter(hist, [b3v * L + lanes], ones, mask=m)
        b3, need = boundary_find(reduce_and_clear_hist(), need)
        p24 = p16 * 256 + b3

        @plsc.parallel_loop(0, nv, unroll=2)
        def _(v):
            sv = cand_s[pl.ds(v * L, L)]
            valid = (v * L + lanes) < cnt
            m = valid & (lax.shift_right_arithmetic(sv, 8) == p24)
            plsc.addupdate_scatter(hist, [(sv & 0xFF) * L + lanes], ones,
                                   mask=m)
        b4, need_final = boundary_find(reduce_and_clear_hist(), need)
        t_s = p24 * 256 + b4

        @pl.when(not_first)
        def _():
            pltpu.make_async_copy(crow, codes_hbm.at[row - 2], sem).wait()
            for j in range(K // L):
                zi = selref[pl.ds(j * L, L)]
                plsc.store_scatter(crow, [zi], zf)

        @plsc.parallel_loop(0, nv, unroll=2,
                            carry=(jnp.int32(0), jnp.int32(0)))
        def _(v, carry):
            pos2, eqc = carry
            sv = cand_s[pl.ds(v * L, L)]
            iv = cand_i[pl.ds(v * L, L)]
            valid = (v * L + lanes) < cnt
            gt = valid & (sv > t_s)
            eq = valid & (sv == t_s)
            eqi = eq.astype(jnp.int32)
            rank = eqc + jnp.cumsum(eqi) - eqi
            take = gt | (eq & (rank < need_final))
            fv = lax.bitcast_convert_type(
                sv ^ (lax.shift_right_arithmetic(sv, 31)
                      & jnp.int32(0x7FFFFFFF)),
                jnp.float32)
            val = jnp.maximum(fv, 0.0)
            plsc.store_scatter(crow, [iv], val, mask=take)
            plsc.store_compressed(selref.at[pl.ds(pos2, L)], iv, mask=take)
            return (pos2 + popcnt(take), eqc + popcnt(eq))
        pltpu.async_copy(crow, codes_hbm.at[row], sem)

    @plsc.parallel_loop(0, NVREG, unroll=UN)
    def _(i):
        crow_a[pl.ds(i * L, L)] = zf
        crow_b[pl.ds(i * L, L)] = zf

    @plsc.parallel_loop(0, 256, unroll=UN)
    def _(i):
        hist[pl.ds(i * L, L)] = zi16

    pltpu.async_copy(h_hbm.at[base], hrow_a, sem_ha)

    def pair_body(i, _):
        row0 = base + 2 * i
        pltpu.async_copy(h_hbm.at[row0 + 1], hrow_b, sem_hb)
        pltpu.make_async_copy(h_hbm.at[row0], hrow_a, sem_ha).wait()
        work(row0, hrow_a, crow_a, sel_a, sem_a, i >= 1)

        @pl.when(i < rows_per_w // 2 - 1)
        def _():
            pltpu.async_copy(h_hbm.at[row0 + 2], hrow_a, sem_ha)

        pltpu.make_async_copy(h_hbm.at[row0 + 1], hrow_b, sem_hb).wait()
        work(row0 + 1, hrow_b, crow_b, sel_b, sem_b, i >= 1)
        return 0
    lax.fori_loop(0, rows_per_w // 2, pair_body, 0)

    pltpu.make_async_copy(
        crow_a, codes_hbm.at[base + rows_per_w - 2], sem_a).wait()
    pltpu.make_async_copy(
        crow_b, codes_hbm.at[base + rows_per_w - 1], sem_b).wait()


def _sc_topk_codes(h):
    ntok = h.shape[0]
    mesh = plsc.VectorSubcoreMesh(
        core_axis_name="c", subcore_axis_name="s", num_cores=2)
    f = pl.kernel(
        functools.partial(_sc_body, ntok // NW),
        out_type=jax.ShapeDtypeStruct((ntok, WIDTH), jnp.float32),
        mesh=mesh,
        compiler_params=pltpu.CompilerParams(needs_layout_passes=False),
        scratch_types=[
            pltpu.VMEM((WIDTH,), jnp.float32),
            pltpu.VMEM((WIDTH,), jnp.float32),
            pltpu.VMEM((WIDTH + L,), jnp.int32),
            pltpu.VMEM((WIDTH + L,), jnp.int32),
            pltpu.VMEM((256 * L,), jnp.int32),
            pltpu.VMEM((256,), jnp.int32),
            pltpu.VMEM((K + L,), jnp.int32),
            pltpu.VMEM((K + L,), jnp.int32),
            pltpu.VMEM((WIDTH,), jnp.float32),
            pltpu.VMEM((WIDTH,), jnp.float32),
            pltpu.SemaphoreType.DMA,
            pltpu.SemaphoreType.DMA,
            pltpu.SemaphoreType.DMA,
            pltpu.SemaphoreType.DMA,
        ],
    )
    return f(h)


NCHUNK = 8


def _enc_chunk(xc, Ae, be, n):
    bm, bn = 256, 1024
    return pl.pallas_call(
        _enc_body,
        grid=(n // bm, WIDTH // bn),
        in_specs=[
            pl.BlockSpec((bm, INPUT_DIM), lambda i, j: (i, 0)),
            pl.BlockSpec((bn, INPUT_DIM), lambda i, j: (j, 0)),
            pl.BlockSpec((1, bn), lambda i, j: (0, j)),
        ],
        out_specs=pl.BlockSpec((bm, bn), lambda i, j: (i, j)),
        out_shape=jax.ShapeDtypeStruct((n, WIDTH), jnp.float32),
    )(xc, Ae, be)


def _dec_chunk(codes, Ad, bd, n):
    bm2, bn2, bk2 = 512, 1024, 2048
    return pl.pallas_call(
        _dec_body,
        grid=(n // bm2, INPUT_DIM // bn2, WIDTH // bk2),
        in_specs=[
            pl.BlockSpec((bm2, bk2), lambda i, j, k: (i, k)),
            pl.BlockSpec((bn2, bk2), lambda i, j, k: (j, k)),
            pl.BlockSpec((1, bn2), lambda i, j, k: (0, j)),
        ],
        out_specs=pl.BlockSpec((bm2, bn2), lambda i, j, k: (i, j)),
        out_shape=jax.ShapeDtypeStruct((n, INPUT_DIM), jnp.float32),
        compiler_params=pltpu.CompilerParams(
            dimension_semantics=("parallel", "parallel", "arbitrary"),
        ),
    )(codes, Ad, bd)


@jax.jit
def kernel(x, Ae, be, bd, Ad):
    n = x.shape[0]
    xc = x - bd
    nc = n // NCHUNK
    outs = []
    for ci in range(NCHUNK):
        xci = lax.slice_in_dim(xc, ci * nc, (ci + 1) * nc, axis=0)
        h = _enc_chunk(xci, Ae, be, nc)
        codes = _sc_topk_codes(h)
        outs.append(_dec_chunk(codes, Ad, bd, nc))
    return jnp.concatenate(outs, axis=0)
